# baseline (speedup 1.0000x reference)
import jax
import jax.numpy as jnp
from jax.experimental import pallas as pl
from jax.experimental.pallas import tpu as pltpu

_DIM = 4096
_PROBE_DIM = 128
_ROWS = 256
_NBUF = 5


def _bucket_kernel(nstep, x_hbm, p_ref, out_ref, buf, sem):
    half = nstep // 2

    def chunk_of(i):
        return jax.lax.rem(i, 2) * half + i // 2

    def start_copy(i):
        slot = jax.lax.rem(i, _NBUF)
        pltpu.make_async_copy(
            x_hbm.at[pl.ds(chunk_of(i) * _ROWS, _ROWS), :],
            buf.at[slot],
            sem.at[slot],
        ).start()

    for s in range(_NBUF - 1):
        start_copy(s)

    def step(i, carry):
        @pl.when(i + _NBUF - 1 < nstep)
        def _():
            start_copy(i + _NBUF - 1)

        slot = jax.lax.rem(i, _NBUF)
        pltpu.make_async_copy(
            x_hbm.at[pl.ds(chunk_of(i) * _ROWS, _ROWS), :],
            buf.at[slot],
            sem.at[slot],
        ).wait()
        proj = jnp.dot(buf[slot], p_ref[...],
                       preferred_element_type=jnp.float32)
        m = jnp.max(proj, axis=1, keepdims=True)
        col = jax.lax.broadcasted_iota(jnp.int32, proj.shape, 1)
        idx = jnp.min(jnp.where(proj == m, col, _PROBE_DIM), axis=1)
        out_ref[chunk_of(i)] = idx.reshape(1, _ROWS)

        return carry

    jax.lax.fori_loop(0, nstep, step, 0)


def kernel(x, probe):
    b, s, d = x.shape
    n = b * s
    nstep = n // _ROWS
    xf = x.reshape(n, d)
    out = pl.pallas_call(
        lambda *refs: _bucket_kernel(nstep, *refs),
        in_specs=[
            pl.BlockSpec(memory_space=pl.ANY),
            pl.BlockSpec(memory_space=pltpu.VMEM),
        ],
        out_specs=pl.BlockSpec(memory_space=pltpu.VMEM),
        out_shape=jax.ShapeDtypeStruct((nstep, 1, _ROWS), jnp.int32),
        scratch_shapes=[
            pltpu.VMEM((_NBUF, _ROWS, _DIM), jnp.float32),
            pltpu.SemaphoreType.DMA((_NBUF,)),
        ],
    )(xf, probe)
    return out.reshape(b, s)


# speedup vs baseline: 1.0107x; 1.0107x over previous
import jax
import jax.numpy as jnp
from jax.experimental import pallas as pl
from jax.experimental.pallas import tpu as pltpu

_DIM = 4096
_PROBE_DIM = 128
_ROWS = 256
_NBUF = 3


def _bucket_kernel(nstep, x_hbm, p_ref, out_ref, buf, sem):
    half = nstep // 2

    def chunk_of(i):
        return jax.lax.rem(i, 2) * half + i // 2

    def start_copy(i):
        slot = jax.lax.rem(i, _NBUF)
        pltpu.make_async_copy(
            x_hbm.at[pl.ds(chunk_of(i) * _ROWS, _ROWS), :],
            buf.at[slot],
            sem.at[slot],
        ).start()

    for s in range(_NBUF):
        start_copy(s)

    def step(i, carry):
        slot = jax.lax.rem(i, _NBUF)
        pltpu.make_async_copy(
            x_hbm.at[pl.ds(chunk_of(i) * _ROWS, _ROWS), :],
            buf.at[slot],
            sem.at[slot],
        ).wait()
        proj = jnp.dot(buf[slot], p_ref[...],
                       preferred_element_type=jnp.float32)
        m = jnp.max(proj, axis=1, keepdims=True)
        col = jax.lax.broadcasted_iota(jnp.int32, proj.shape, 1)
        idx = jnp.min(jnp.where(proj == m, col, _PROBE_DIM), axis=1)
        out_ref[chunk_of(i)] = idx.reshape(1, _ROWS)

        @pl.when(i + _NBUF < nstep)
        def _():
            start_copy(i + _NBUF)

        return carry

    jax.lax.fori_loop(0, nstep, step, 0)


def kernel(x, probe):
    b, s, d = x.shape
    n = b * s
    nstep = n // _ROWS
    xf = x.reshape(n, d)
    out = pl.pallas_call(
        lambda *refs: _bucket_kernel(nstep, *refs),
        in_specs=[
            pl.BlockSpec(memory_space=pl.ANY),
            pl.BlockSpec(memory_space=pltpu.VMEM),
        ],
        out_specs=pl.BlockSpec(memory_space=pltpu.VMEM),
        out_shape=jax.ShapeDtypeStruct((nstep, 1, _ROWS), jnp.int32),
        scratch_shapes=[
            pltpu.VMEM((_NBUF, _ROWS, _DIM), jnp.float32),
            pltpu.SemaphoreType.DMA((_NBUF,)),
        ],
    )(xf, probe)
    return out.reshape(b, s)
